# baseline (speedup 1.0000x reference)
import dataclasses
import functools

import jax
import jax.numpy as jnp
from jax import lax
from jax.experimental import pallas as pl
from jax.experimental.pallas import tpu as pltpu
from jax.experimental.pallas import tpu_sc as plsc

BIN_NUM = 20
BIN_WIDTH = 0.1
FEAT = 96
HALF = BIN_NUM // 2

NC = 2
NS = 16
L = 16
NW = NC * NS

ROWS = 384
COLS = 384
N_PLANES = 4 * FEAT
PPW = N_PLANES // NW
BLK_ROWS = 64
BLKS_PER_PLANE = ROWS // BLK_ROWS
NCH = PPW * BLKS_PER_PLANE
CHUNK = BLK_ROWS * COLS
NCH = PPW * BLKS_PER_PLANE


@jax.jit
def _sc_mtlu(x3, wflat, bflat):
    mesh = plsc.VectorSubcoreMesh(core_axis_name="c", subcore_axis_name="s")
    cp = pltpu.CompilerParams()
    if "needs_layout_passes" in pltpu.CompilerParams.__dataclass_fields__:
        cp = dataclasses.replace(cp, needs_layout_passes=False)

    @functools.partial(
        pl.kernel,
        compiler_params=cp,
        out_type=jax.ShapeDtypeStruct(x3.shape, jnp.float32),
        mesh=mesh,
        scratch_types=[
            pltpu.VMEM((FEAT * BIN_NUM,), jnp.float32),
            pltpu.VMEM((FEAT * BIN_NUM,), jnp.float32),
            pltpu.VMEM((BLK_ROWS, COLS), jnp.float32),
            pltpu.VMEM((BLK_ROWS, COLS), jnp.float32),
            pltpu.VMEM((BLK_ROWS, COLS), jnp.float32),
            pltpu.VMEM((BLK_ROWS, COLS), jnp.float32),
            pltpu.SemaphoreType.DMA,
            pltpu.SemaphoreType.DMA,
            pltpu.SemaphoreType.DMA,
            pltpu.SemaphoreType.DMA,
        ],
    )
    def k(x_hbm, w_hbm, b_hbm, o_hbm, tw, tb,
          xin0, xin1, xout0, xout1, si0, si1, so0, so1):
        wid = lax.axis_index("s") * NC + lax.axis_index("c")
        pltpu.sync_copy(w_hbm, tw)
        pltpu.sync_copy(b_hbm, tb)
        xin = (xin0, xin1)
        xout = (xout0, xout1)
        sin = (si0, si1)
        sout = (so0, so1)

        def block_at(t):
            plane = wid * PPW + t // BLKS_PER_PLANE
            c20 = (plane % FEAT) * BIN_NUM
            r0 = (t % BLKS_PER_PLANE) * BLK_ROWS
            return plane, r0, c20

        def start_in(t, b):
            plane, r0, _ = block_at(t)
            pltpu.async_copy(
                x_hbm.at[plane, pl.ds(r0, BLK_ROWS)], xin[b], sin[b])

        start_in(0, 0)

        @pl.loop(0, NCH, step=2)
        def _pair(tt):
            for b in range(2):
                t = tt + b
                plane, r0, c20 = block_at(t)

                @pl.when(t + 1 < NCH)
                def _prefetch():
                    start_in(t + 1, 1 - b)

                pltpu.make_async_copy(
                    x_hbm.at[plane, pl.ds(r0, BLK_ROWS)], xin[b],
                    sin[b]).wait()

                @pl.when(t >= 2)
                def _drain():
                    pltpu.make_async_copy(
                        xout[b], o_hbm.at[plane, pl.ds(r0, BLK_ROWS)],
                        sout[b]).wait()

                src = xin[b]
                dst = xout[b]

                @plsc.parallel_loop(0, BLK_ROWS, step=1, unroll=2)
                def _row(r):
                    for kk in range(COLS // L):
                        xv = src[r, pl.ds(kk * L, L)]
                        f = (xv * jnp.float32(1.0 / BIN_WIDTH)
                             + jnp.float32(HALF))
                        f = jnp.minimum(jnp.maximum(f, jnp.float32(0.0)),
                                        jnp.float32(BIN_NUM - 0.5))
                        idx = f.astype(jnp.int32) + c20
                        wv = plsc.load_gather(tw, [idx])
                        bv = plsc.load_gather(tb, [idx])
                        dst[r, pl.ds(kk * L, L)] = wv * xv + bv

                pltpu.async_copy(
                    xout[b], o_hbm.at[plane, pl.ds(r0, BLK_ROWS)], sout[b])

        for b in range(2):
            plane, r0, _ = block_at(NCH - 2 + b)
            pltpu.make_async_copy(
                xout[b], o_hbm.at[plane, pl.ds(r0, BLK_ROWS)],
                sout[b]).wait()

    return k(x3, wflat, bflat)


def kernel(x, mtlu_y, mtlu_y_shift):
    index = jnp.arange(-HALF + 1, HALF + 1, dtype=jnp.float32)
    weight = (mtlu_y - mtlu_y_shift) / BIN_WIDTH
    bias = mtlu_y - (mtlu_y - mtlu_y_shift) * index
    x3 = x.reshape(N_PLANES, ROWS, COLS)
    out = _sc_mtlu(x3, weight.reshape(-1), bias.reshape(-1))
    return out.reshape(x.shape)


# speedup vs baseline: 1.1349x; 1.1349x over previous
import dataclasses
import functools

import jax
import jax.numpy as jnp
from jax import lax
from jax.experimental import pallas as pl
from jax.experimental.pallas import tpu as pltpu
from jax.experimental.pallas import tpu_sc as plsc

BIN_NUM = 20
BIN_WIDTH = 0.1
FEAT = 96
HALF = BIN_NUM // 2
BIN_PAD = 24

NC = 2
NS = 16
L = 16
NW = NC * NS

ROWS = 384
COLS = 384
N_PLANES = 4 * FEAT
PPW = N_PLANES // NW
BLK_ROWS = 64
BLKS_PER_PLANE = ROWS // BLK_ROWS
NCH = PPW * BLKS_PER_PLANE


@jax.jit
def _sc_mtlu(x3, wflat, bflat):
    mesh = plsc.VectorSubcoreMesh(core_axis_name="c", subcore_axis_name="s")
    cp = pltpu.CompilerParams()
    if "needs_layout_passes" in pltpu.CompilerParams.__dataclass_fields__:
        cp = dataclasses.replace(cp, needs_layout_passes=False)

    @functools.partial(
        pl.kernel,
        compiler_params=cp,
        out_type=jax.ShapeDtypeStruct(x3.shape, jnp.float32),
        mesh=mesh,
        scratch_types=[
            pltpu.VMEM((FEAT * BIN_PAD,), jnp.float32),
            pltpu.VMEM((FEAT * BIN_PAD,), jnp.float32),
            pltpu.VMEM((BLK_ROWS, COLS), jnp.float32),
            pltpu.VMEM((BLK_ROWS, COLS), jnp.float32),
            pltpu.VMEM((BLK_ROWS, COLS), jnp.float32),
            pltpu.VMEM((BLK_ROWS, COLS), jnp.float32),
            pltpu.SemaphoreType.DMA,
            pltpu.SemaphoreType.DMA,
            pltpu.SemaphoreType.DMA,
            pltpu.SemaphoreType.DMA,
        ],
    )
    def k(x_hbm, w_hbm, b_hbm, o_hbm, tw, tb,
          xin0, xin1, xout0, xout1, si0, si1, so0, so1):
        wid = lax.axis_index("s") * NC + lax.axis_index("c")
        pltpu.sync_copy(w_hbm, tw)
        pltpu.sync_copy(b_hbm, tb)
        xin = (xin0, xin1)
        xout = (xout0, xout1)
        sin = (si0, si1)
        sout = (so0, so1)

        def block_at(t):
            plane = wid * PPW + t // BLKS_PER_PLANE
            c20 = (plane % FEAT) * BIN_PAD
            r0 = (t % BLKS_PER_PLANE) * BLK_ROWS
            return plane, r0, c20

        def start_in(t, b):
            plane, r0, _ = block_at(t)
            pltpu.async_copy(
                x_hbm.at[plane, pl.ds(r0, BLK_ROWS)], xin[b], sin[b])

        start_in(0, 0)

        @pl.loop(0, NCH, step=2)
        def _pair(tt):
            for b in range(2):
                t = tt + b
                plane, r0, c20 = block_at(t)

                @pl.when(t + 1 < NCH)
                def _prefetch():
                    start_in(t + 1, 1 - b)

                pltpu.make_async_copy(
                    x_hbm.at[plane, pl.ds(r0, BLK_ROWS)], xin[b],
                    sin[b]).wait()

                @pl.when(t >= 2)
                def _drain():
                    pltpu.make_async_copy(
                        xout[b], o_hbm.at[plane, pl.ds(r0, BLK_ROWS)],
                        sout[b]).wait()

                src = xin[b]
                dst = xout[b]
                twc = tw.at[pl.ds(c20, BIN_NUM)]
                tbc = tb.at[pl.ds(c20, BIN_NUM)]

                @pl.loop(0, BLK_ROWS)
                def _row(r):
                    @plsc.parallel_loop(0, COLS, step=L, unroll=8)
                    def _vec(i):
                        xv = src[r, pl.ds(i, L)]
                        f = (xv * jnp.float32(1.0 / BIN_WIDTH)
                             + jnp.float32(HALF))
                        f = jnp.minimum(jnp.maximum(f, jnp.float32(0.0)),
                                        jnp.float32(BIN_NUM - 0.5))
                        idx = f.astype(jnp.int32)
                        wv = plsc.load_gather(twc, [idx])
                        bv = plsc.load_gather(tbc, [idx])
                        dst[r, pl.ds(i, L)] = wv * xv + bv

                pltpu.async_copy(
                    xout[b], o_hbm.at[plane, pl.ds(r0, BLK_ROWS)], sout[b])

        for b in range(2):
            plane, r0, _ = block_at(NCH - 2 + b)
            pltpu.make_async_copy(
                xout[b], o_hbm.at[plane, pl.ds(r0, BLK_ROWS)],
                sout[b]).wait()

    return k(x3, wflat, bflat)


def kernel(x, mtlu_y, mtlu_y_shift):
    index = jnp.arange(-HALF + 1, HALF + 1, dtype=jnp.float32)
    weight = (mtlu_y - mtlu_y_shift) / BIN_WIDTH
    bias = mtlu_y - (mtlu_y - mtlu_y_shift) * index
    pad = ((0, 0), (0, BIN_PAD - BIN_NUM))
    weight = jnp.pad(weight, pad)
    bias = jnp.pad(bias, pad)
    x3 = x.reshape(N_PLANES, ROWS, COLS)
    out = _sc_mtlu(x3, weight.reshape(-1), bias.reshape(-1))
    return out.reshape(x.shape)


# speedup vs baseline: 1.1735x; 1.0340x over previous
import dataclasses
import functools

import jax
import jax.numpy as jnp
from jax import lax
from jax.experimental import pallas as pl
from jax.experimental.pallas import tpu as pltpu
from jax.experimental.pallas import tpu_sc as plsc

BIN_NUM = 20
BIN_WIDTH = 0.1
FEAT = 96
HALF = BIN_NUM // 2
BIN_PAD = 24

NC = 2
NS = 16
L = 16
NW = NC * NS

ROWS = 384
COLS = 384
N_PLANES = 4 * FEAT
PPW = N_PLANES // NW
BLK_ROWS = 64
BLKS_PER_PLANE = ROWS // BLK_ROWS
NCH = PPW * BLKS_PER_PLANE


@jax.jit
def _sc_mtlu(x3, wflat, bflat):
    mesh = plsc.VectorSubcoreMesh(core_axis_name="c", subcore_axis_name="s")
    cp = pltpu.CompilerParams()
    if "needs_layout_passes" in pltpu.CompilerParams.__dataclass_fields__:
        cp = dataclasses.replace(cp, needs_layout_passes=False)

    @functools.partial(
        pl.kernel,
        compiler_params=cp,
        out_type=jax.ShapeDtypeStruct(x3.shape, jnp.float32),
        mesh=mesh,
        scratch_types=[
            pltpu.VMEM((FEAT * BIN_PAD,), jnp.float32),
            pltpu.VMEM((FEAT * BIN_PAD,), jnp.float32),
            pltpu.VMEM((BLK_ROWS, COLS), jnp.float32),
            pltpu.VMEM((BLK_ROWS, COLS), jnp.float32),
            pltpu.VMEM((BLK_ROWS, COLS), jnp.float32),
            pltpu.VMEM((BLK_ROWS, COLS), jnp.float32),
            pltpu.SemaphoreType.DMA,
            pltpu.SemaphoreType.DMA,
            pltpu.SemaphoreType.DMA,
            pltpu.SemaphoreType.DMA,
        ],
    )
    def k(x_hbm, w_hbm, b_hbm, o_hbm, tw, tb,
          xin0, xin1, xout0, xout1, si0, si1, so0, so1):
        wid = lax.axis_index("s") * NC + lax.axis_index("c")
        pltpu.sync_copy(w_hbm, tw)
        pltpu.sync_copy(b_hbm, tb)
        xin = (xin0, xin1)
        xout = (xout0, xout1)
        sin = (si0, si1)
        sout = (so0, so1)

        def block_at(t):
            plane = wid * PPW + t // BLKS_PER_PLANE
            c20 = (plane % FEAT) * BIN_PAD
            r0 = (t % BLKS_PER_PLANE) * BLK_ROWS
            return plane, r0, c20

        def start_in(t, b):
            plane, r0, _ = block_at(t)
            pltpu.async_copy(
                x_hbm.at[plane, pl.ds(r0, BLK_ROWS)], xin[b], sin[b])

        start_in(0, 0)

        @pl.loop(0, NCH, step=2)
        def _pair(tt):
            for b in range(2):
                t = tt + b
                plane, r0, c20 = block_at(t)

                @pl.when(t + 1 < NCH)
                def _prefetch():
                    start_in(t + 1, 1 - b)

                pltpu.make_async_copy(
                    x_hbm.at[plane, pl.ds(r0, BLK_ROWS)], xin[b],
                    sin[b]).wait()

                @pl.when(t >= 2)
                def _drain():
                    pltpu.make_async_copy(
                        xout[b], o_hbm.at[plane, pl.ds(r0, BLK_ROWS)],
                        sout[b]).wait()

                src = xin[b]
                dst = xout[b]
                twc = tw.at[pl.ds(c20, BIN_NUM)]
                tbc = tb.at[pl.ds(c20, BIN_NUM)]

                @plsc.parallel_loop(0, BLK_ROWS, step=1, unroll=1)
                def _row(r):
                    for kk in range(COLS // L):
                        i = kk * L
                        xv = src[r, pl.ds(i, L)]
                        f = (xv * jnp.float32(1.0 / BIN_WIDTH)
                             + jnp.float32(HALF))
                        f = jnp.minimum(jnp.maximum(f, jnp.float32(0.0)),
                                        jnp.float32(BIN_NUM - 0.5))
                        idx = f.astype(jnp.int32)
                        wv = plsc.load_gather(twc, [idx])
                        bv = plsc.load_gather(tbc, [idx])
                        dst[r, pl.ds(i, L)] = wv * xv + bv

                pltpu.async_copy(
                    xout[b], o_hbm.at[plane, pl.ds(r0, BLK_ROWS)], sout[b])

        for b in range(2):
            plane, r0, _ = block_at(NCH - 2 + b)
            pltpu.make_async_copy(
                xout[b], o_hbm.at[plane, pl.ds(r0, BLK_ROWS)],
                sout[b]).wait()

    return k(x3, wflat, bflat)


def kernel(x, mtlu_y, mtlu_y_shift):
    index = jnp.arange(-HALF + 1, HALF + 1, dtype=jnp.float32)
    weight = (mtlu_y - mtlu_y_shift) / BIN_WIDTH
    bias = mtlu_y - (mtlu_y - mtlu_y_shift) * index
    pad = ((0, 0), (0, BIN_PAD - BIN_NUM))
    weight = jnp.pad(weight, pad)
    bias = jnp.pad(bias, pad)
    x3 = x.reshape(N_PLANES, ROWS, COLS)
    out = _sc_mtlu(x3, weight.reshape(-1), bias.reshape(-1))
    return out.reshape(x.shape)
